# baseline (speedup 1.0000x reference)
import jax
import jax.numpy as jnp
from jax.experimental import pallas as pl
from jax.experimental.pallas import tpu as pltpu

B = 4096
D_IN = 128
ENC = 64
PROJ_H = 128
PROJ_D = 128
QSIZE = 4096
TEMP = 0.07
HW = 2.0
HT = 0.75
EPS = 1e-5
KK = max(1, int(0.1 * QSIZE))

ROWS = 256
NBLK = B // ROWS


def _prologue_kernel(x1, x2, y1, y2, W_enc, b_enc, Wuf, buf,
                     W1, b1, g1, be1, W2, b2, g2, be2,
                     q_out, pos_out, pinn_out):
    h1 = jnp.tanh(jnp.dot(x1[...], W_enc[...]) + b_enc[...])
    h2 = jnp.tanh(jnp.dot(x2[...], W_enc[...]) + b_enc[...])

    uf1 = jnp.dot(h1, Wuf[...]) + buf[...]
    uf2 = jnp.dot(h2, Wuf[...]) + buf[...]
    u1 = uf1[:, 0:1]
    f1 = uf1[:, 1:2]
    u2 = uf2[:, 0:1]
    f2 = uf2[:, 1:2]

    inv_b = 1.0 / B
    data_loss = 0.5 * (jnp.sum((u1 - y1[...]) ** 2) * inv_b
                       + jnp.sum((u2 - y2[...]) ** 2) * inv_b)
    pde_loss = 0.5 * (jnp.sum(f1 * f1) * inv_b + jnp.sum(f2 * f2) * inv_b)
    physics_loss = jnp.sum(
        jnp.maximum((u2 - u1) * (y1[...] - y2[...]), 0.0)) * inv_b
    pinn = data_loss + pde_loss + physics_loss
    pinn_out[...] = pinn.reshape(1, 1)

    def proj(h):
        a1 = jnp.dot(h, W1[...]) + b1[...]
        mu1 = jnp.mean(a1, axis=0, keepdims=True)
        c1 = a1 - mu1
        var1 = jnp.mean(c1 * c1, axis=0, keepdims=True)
        r1 = jnp.maximum(g1[...] * c1 * jax.lax.rsqrt(var1 + EPS) + be1[...],
                         0.0)
        a2 = jnp.dot(r1, W2[...]) + b2[...]
        mu2 = jnp.mean(a2, axis=0, keepdims=True)
        c2 = a2 - mu2
        var2 = jnp.mean(c2 * c2, axis=0, keepdims=True)
        z = g2[...] * c2 * jax.lax.rsqrt(var2 + EPS) + be2[...]
        n = jnp.sqrt(jnp.sum(z * z, axis=1, keepdims=True))
        return z / jnp.maximum(n, 1e-12)

    q = proj(h1)
    kvec = proj(h2)
    q_out[...] = q
    pos_out[...] = jnp.sum(q * kvec, axis=1, keepdims=True)


def _neg_kernel(q, pos_t, queue, lse_h_out, lse_f_out, pos_out, cnt_out):
    i = pl.program_id(0)
    neg = jax.lax.dot_general(
        queue[...].astype(jnp.bfloat16), q[...].astype(jnp.bfloat16),
        (((1,), (1,)), ((), ())),
        preferred_element_type=jnp.float32)

    bits = jax.lax.bitcast_convert_type(neg, jnp.int32)
    keys = bits ^ ((bits >> 31) & jnp.int32(0x7FFFFFFF))

    hi = (keys >> 16).astype(jnp.int16)

    def _count_ge(data16, cand):
        m16 = (data16 >= cand.astype(jnp.int16)).astype(jnp.int16)
        v = pltpu.bitcast(m16, jnp.int32)
        n = v.shape[0]
        while n > 64:
            n //= 2
            v = v[:n] + v[n:]
        s = jnp.sum(v, axis=0, keepdims=True)
        return (s & jnp.int32(0xFFFF)) + (s >> 16)

    def body1(j, res):
        bit = jax.lax.shift_left(jnp.int32(1), jnp.int32(13) - j)
        cand = res + bit
        return jnp.where(_count_ge(hi, cand) >= KK, cand, res)

    cnt_pos = _count_ge(hi, jnp.zeros((1, ROWS), jnp.int32))
    res1_0 = jnp.where(cnt_pos >= KK, jnp.int32(0), jnp.int32(-16384))
    t16 = jax.lax.fori_loop(0, 14, body1, res1_0)

    c_above = _count_ge(hi, t16 + 1)
    k2 = KK - c_above

    low = keys.astype(jnp.int16) ^ jnp.int16(-0x8000)
    w = jnp.where(hi == t16.astype(jnp.int16), low, jnp.int16(-0x8000))

    def body2(j, res):
        bit = jax.lax.shift_left(jnp.int32(1), jnp.int32(15) - j)
        cand = res + bit
        return jnp.where(_count_ge(w, cand) >= k2, cand, res)

    res2_0 = jnp.full((1, ROWS), -32768, jnp.int32)
    low_k = jax.lax.fori_loop(0, 16, body2, res2_0)

    res = (jax.lax.shift_left(t16, 16)
           | ((low_k ^ jnp.int32(0x8000)) & jnp.int32(0xFFFF)))
    vk_bits = res ^ ((res >> 31) & jnp.int32(0x7FFFFFFF))
    vk = jax.lax.bitcast_convert_type(vk_bits, jnp.float32)

    inv_t = 1.0 / TEMP
    e = jnp.exp(neg * inv_t)
    e2 = e * e
    is_hard = neg > HT
    s_hard = jnp.sum(jnp.where(is_hard, e2, e), axis=0, keepdims=True)
    s_fb = jnp.sum(jnp.where(neg >= vk, e2, e), axis=0, keepdims=True)

    p = pos_t[...]
    ep = jnp.exp(p * inv_t)
    lse_h = jnp.sum(jnp.log(s_hard + ep))
    lse_f = jnp.sum(jnp.log(s_fb + ep))
    psum = jnp.sum(p)
    hcnt = jnp.sum(is_hard.astype(jnp.float32))

    @pl.when(i == 0)
    def _():
        lse_h_out[...] = jnp.zeros_like(lse_h_out)
        lse_f_out[...] = jnp.zeros_like(lse_f_out)
        pos_out[...] = jnp.zeros_like(pos_out)
        cnt_out[...] = jnp.zeros_like(cnt_out)

    lse_h_out[...] += lse_h.reshape(1, 1)
    lse_f_out[...] += lse_f.reshape(1, 1)
    pos_out[...] += psum.reshape(1, 1)
    cnt_out[...] += hcnt.reshape(1, 1)


@jax.jit
def kernel(x1, x2, y1, y2, W_enc, b_enc, W_u, b_u, W_f, b_f,
           W1, b1, g1, be1, W2, b2, g2, be2, queue):
    f32 = jnp.float32
    Wuf = jnp.concatenate([W_u, W_f], axis=1)
    buf = jnp.concatenate([b_u, b_f]).reshape(1, 2)

    q, pos, pinn = pl.pallas_call(
        _prologue_kernel,
        out_shape=(
            jax.ShapeDtypeStruct((B, PROJ_D), f32),
            jax.ShapeDtypeStruct((B, 1), f32),
            jax.ShapeDtypeStruct((1, 1), f32),
        ),
    )(x1, x2, y1, y2, W_enc, b_enc.reshape(1, ENC), Wuf, buf,
      W1, b1.reshape(1, PROJ_H), g1.reshape(1, PROJ_H),
      be1.reshape(1, PROJ_H), W2, b2.reshape(1, PROJ_D),
      g2.reshape(1, PROJ_D), be2.reshape(1, PROJ_D))

    pos_t = pos.reshape(1, B)

    lse_h, lse_f, pos_sum, hard_cnt = pl.pallas_call(
        _neg_kernel,
        grid=(NBLK,),
        in_specs=[
            pl.BlockSpec((ROWS, PROJ_D), lambda i: (i, 0)),
            pl.BlockSpec((1, ROWS), lambda i: (0, i)),
            pl.BlockSpec((QSIZE, PROJ_D), lambda i: (0, 0)),
        ],
        out_specs=(
            pl.BlockSpec((1, 1), lambda i: (0, 0)),
            pl.BlockSpec((1, 1), lambda i: (0, 0)),
            pl.BlockSpec((1, 1), lambda i: (0, 0)),
            pl.BlockSpec((1, 1), lambda i: (0, 0)),
        ),
        out_shape=(
            jax.ShapeDtypeStruct((1, 1), f32),
            jax.ShapeDtypeStruct((1, 1), f32),
            jax.ShapeDtypeStruct((1, 1), f32),
            jax.ShapeDtypeStruct((1, 1), f32),
        ),
    )(q, pos_t, queue)

    lse_sum = jnp.where(hard_cnt[0, 0] > 0.0, lse_h[0, 0], lse_f[0, 0])
    contrastive = (lse_sum - pos_sum[0, 0] / TEMP) / B
    pinn_loss = pinn[0, 0]
    total = pinn_loss + contrastive
    return total, pinn_loss, contrastive


# speedup vs baseline: 1.0133x; 1.0133x over previous
import jax
import jax.numpy as jnp
from jax.experimental import pallas as pl
from jax.experimental.pallas import tpu as pltpu

B = 4096
D_IN = 128
ENC = 64
PROJ_H = 128
PROJ_D = 128
QSIZE = 4096
TEMP = 0.07
HW = 2.0
HT = 0.75
EPS = 1e-5
KK = max(1, int(0.1 * QSIZE))

ROWS = 256
NBLK = B // ROWS


def _prologue_kernel(x1, x2, y1, y2, W_enc, b_enc, Wuf, buf,
                     W1, b1, g1, be1, W2, b2, g2, be2,
                     q_out, pos_out, pinn_out):
    h1 = jnp.tanh(jnp.dot(x1[...], W_enc[...]) + b_enc[...])
    h2 = jnp.tanh(jnp.dot(x2[...], W_enc[...]) + b_enc[...])

    uf1 = jnp.dot(h1, Wuf[...]) + buf[...]
    uf2 = jnp.dot(h2, Wuf[...]) + buf[...]
    u1 = uf1[:, 0:1]
    f1 = uf1[:, 1:2]
    u2 = uf2[:, 0:1]
    f2 = uf2[:, 1:2]

    inv_b = 1.0 / B
    data_loss = 0.5 * (jnp.sum((u1 - y1[...]) ** 2) * inv_b
                       + jnp.sum((u2 - y2[...]) ** 2) * inv_b)
    pde_loss = 0.5 * (jnp.sum(f1 * f1) * inv_b + jnp.sum(f2 * f2) * inv_b)
    physics_loss = jnp.sum(
        jnp.maximum((u2 - u1) * (y1[...] - y2[...]), 0.0)) * inv_b
    pinn = data_loss + pde_loss + physics_loss
    pinn_out[...] = pinn.reshape(1, 1)

    def proj(h):
        a1 = jnp.dot(h, W1[...]) + b1[...]
        mu1 = jnp.mean(a1, axis=0, keepdims=True)
        c1 = a1 - mu1
        var1 = jnp.mean(c1 * c1, axis=0, keepdims=True)
        r1 = jnp.maximum(g1[...] * c1 * jax.lax.rsqrt(var1 + EPS) + be1[...],
                         0.0)
        a2 = jnp.dot(r1, W2[...]) + b2[...]
        mu2 = jnp.mean(a2, axis=0, keepdims=True)
        c2 = a2 - mu2
        var2 = jnp.mean(c2 * c2, axis=0, keepdims=True)
        z = g2[...] * c2 * jax.lax.rsqrt(var2 + EPS) + be2[...]
        n = jnp.sqrt(jnp.sum(z * z, axis=1, keepdims=True))
        return z / jnp.maximum(n, 1e-12)

    q = proj(h1)
    kvec = proj(h2)
    q_out[...] = q
    pos_out[...] = jnp.sum(q * kvec, axis=1, keepdims=True)


def _neg_kernel(q, pos_t, queue, lse_h_out, lse_f_out, pos_out, cnt_out):
    i = pl.program_id(0)
    neg = jax.lax.dot_general(
        queue[...].astype(jnp.bfloat16), q[...].astype(jnp.bfloat16),
        (((1,), (1,)), ((), ())),
        preferred_element_type=jnp.float32)

    bits = jax.lax.bitcast_convert_type(neg, jnp.int32)
    keys = bits ^ ((bits >> 31) & jnp.int32(0x7FFFFFFF))

    hi = (keys >> 16).astype(jnp.int16)

    def _count_ge(data16, cand):
        m16 = (data16 >= cand.astype(jnp.int16)).astype(jnp.int16)
        m32 = pltpu.bitcast(m16, jnp.int32)
        s = jnp.sum(m32, axis=0, keepdims=True)
        return (s & jnp.int32(0xFFFF)) + (s >> 16)

    def body1(j, res):
        bit = jax.lax.shift_left(jnp.int32(1), jnp.int32(13) - j)
        cand = res + bit
        return jnp.where(_count_ge(hi, cand) >= KK, cand, res)

    cnt_pos = _count_ge(hi, jnp.zeros((1, ROWS), jnp.int32))
    res1_0 = jnp.where(cnt_pos >= KK, jnp.int32(0), jnp.int32(-16384))
    t16 = jax.lax.fori_loop(0, 14, body1, res1_0)

    c_above = _count_ge(hi, t16 + 1)
    k2 = KK - c_above

    low = keys.astype(jnp.int16) ^ jnp.int16(-0x8000)
    w = jnp.where(hi == t16.astype(jnp.int16), low, jnp.int16(-0x8000))

    def body2(j, res):
        bit = jax.lax.shift_left(jnp.int32(1), jnp.int32(15) - j)
        cand = res + bit
        return jnp.where(_count_ge(w, cand) >= k2, cand, res)

    res2_0 = jnp.full((1, ROWS), -32768, jnp.int32)
    low_k = jax.lax.fori_loop(0, 16, body2, res2_0)

    res = (jax.lax.shift_left(t16, 16)
           | ((low_k ^ jnp.int32(0x8000)) & jnp.int32(0xFFFF)))
    vk_bits = res ^ ((res >> 31) & jnp.int32(0x7FFFFFFF))
    vk = jax.lax.bitcast_convert_type(vk_bits, jnp.float32)

    inv_t = 1.0 / TEMP
    e = jnp.exp(neg * inv_t)
    e2 = e * e
    is_hard = neg > HT
    s_hard = jnp.sum(jnp.where(is_hard, e2, e), axis=0, keepdims=True)
    s_fb = jnp.sum(jnp.where(neg >= vk, e2, e), axis=0, keepdims=True)

    p = pos_t[...]
    ep = jnp.exp(p * inv_t)
    lse_h = jnp.sum(jnp.log(s_hard + ep))
    lse_f = jnp.sum(jnp.log(s_fb + ep))
    psum = jnp.sum(p)
    hcnt = jnp.sum(is_hard.astype(jnp.float32))

    @pl.when(i == 0)
    def _():
        lse_h_out[...] = jnp.zeros_like(lse_h_out)
        lse_f_out[...] = jnp.zeros_like(lse_f_out)
        pos_out[...] = jnp.zeros_like(pos_out)
        cnt_out[...] = jnp.zeros_like(cnt_out)

    lse_h_out[...] += lse_h.reshape(1, 1)
    lse_f_out[...] += lse_f.reshape(1, 1)
    pos_out[...] += psum.reshape(1, 1)
    cnt_out[...] += hcnt.reshape(1, 1)


@jax.jit
def kernel(x1, x2, y1, y2, W_enc, b_enc, W_u, b_u, W_f, b_f,
           W1, b1, g1, be1, W2, b2, g2, be2, queue):
    f32 = jnp.float32
    Wuf = jnp.concatenate([W_u, W_f], axis=1)
    buf = jnp.concatenate([b_u, b_f]).reshape(1, 2)

    q, pos, pinn = pl.pallas_call(
        _prologue_kernel,
        out_shape=(
            jax.ShapeDtypeStruct((B, PROJ_D), f32),
            jax.ShapeDtypeStruct((B, 1), f32),
            jax.ShapeDtypeStruct((1, 1), f32),
        ),
    )(x1, x2, y1, y2, W_enc, b_enc.reshape(1, ENC), Wuf, buf,
      W1, b1.reshape(1, PROJ_H), g1.reshape(1, PROJ_H),
      be1.reshape(1, PROJ_H), W2, b2.reshape(1, PROJ_D),
      g2.reshape(1, PROJ_D), be2.reshape(1, PROJ_D))

    pos_t = pos.reshape(1, B)

    lse_h, lse_f, pos_sum, hard_cnt = pl.pallas_call(
        _neg_kernel,
        grid=(NBLK,),
        in_specs=[
            pl.BlockSpec((ROWS, PROJ_D), lambda i: (i, 0)),
            pl.BlockSpec((1, ROWS), lambda i: (0, i)),
            pl.BlockSpec((QSIZE, PROJ_D), lambda i: (0, 0)),
        ],
        out_specs=(
            pl.BlockSpec((1, 1), lambda i: (0, 0)),
            pl.BlockSpec((1, 1), lambda i: (0, 0)),
            pl.BlockSpec((1, 1), lambda i: (0, 0)),
            pl.BlockSpec((1, 1), lambda i: (0, 0)),
        ),
        out_shape=(
            jax.ShapeDtypeStruct((1, 1), f32),
            jax.ShapeDtypeStruct((1, 1), f32),
            jax.ShapeDtypeStruct((1, 1), f32),
            jax.ShapeDtypeStruct((1, 1), f32),
        ),
    )(q, pos_t, queue)

    lse_sum = jnp.where(hard_cnt[0, 0] > 0.0, lse_h[0, 0], lse_f[0, 0])
    contrastive = (lse_sum - pos_sum[0, 0] / TEMP) / B
    pinn_loss = pinn[0, 0]
    total = pinn_loss + contrastive
    return total, pinn_loss, contrastive
